# baseline (speedup 1.0000x reference)
import jax
import jax.numpy as jnp
from jax import lax
from jax.experimental import pallas as pl
from jax.experimental.pallas import tpu as pltpu
from jax.experimental.pallas import tpu_sc as plsc

B, N, C = 8, 16384, 6
K = 1024
L = 16
TPB = 4
BPC = B // 2
NCHUNK = N // TPB
NV = NCHUNK // L
KPT = K // TPB
UNROLL = 16
SLOT = 2 * L
INT_MAX = 2**31 - 1


def _fps_body(pts, out, ch0, ch1, ch2, ch3, ch4, ch5,
              dists, idxc, outb, *comm):
    c = lax.axis_index("c")
    s = lax.axis_index("s")
    gg = s // TPB
    q = s % TPB
    b = c * BPC + gg % BPC
    writer = gg < BPC
    doff = q * NCHUNK

    chans = (ch0, ch1, ch2, ch3, ch4, ch5)
    for ci in range(C):
        pltpu.sync_copy(pts.at[b, ci], chans[ci])

    inf_vec = jnp.full((L,), jnp.inf, jnp.float32)

    def init_body(v, carry):
        dists[pl.ds(v * L, L)] = inf_vec
        return carry

    lax.fori_loop(0, NV, init_body, 0)

    lane = lax.iota(jnp.int32, L)

    def outer(i, far_vec):
        cx = plsc.load_gather(ch0, [far_vec])
        cy = plsc.load_gather(ch1, [far_vec])
        cz = plsc.load_gather(ch2, [far_vec])

        @pl.when((i // KPT) == q)
        def _():
            plsc.store_scatter(idxc, [jnp.full((L,), i % KPT, jnp.int32)],
                               far_vec, mask=(lane == 0))

        best0 = jnp.full((L,), -1.0, jnp.float32)
        bidx0 = jnp.zeros((L,), jnp.int32)

        @plsc.parallel_loop(0, NCHUNK, step=L, unroll=UNROLL,
                            carry=(best0, bidx0))
        def inner(o, carry):
            best, bidx = carry
            xv = ch0[pl.ds(doff + o, L)]
            yv = ch1[pl.ds(doff + o, L)]
            zv = ch2[pl.ds(doff + o, L)]
            dx = xv - cx
            dy = yv - cy
            dz = zv - cz
            t = (dx * dx + dy * dy) + dz * dz
            dv = jnp.minimum(dists[pl.ds(o, L)], t)
            dists[pl.ds(o, L)] = dv
            m = dv > best
            best = jnp.where(m, dv, best)
            bidx = jnp.where(m, doff + o + lane, bidx)
            return best, bidx

        best, bidx = inner

        gmax = jnp.max(best)
        cand = jnp.where(best == gmax, bidx, jnp.full((L,), INT_MAX, jnp.int32))
        lwin_vec = jnp.full((L,), jnp.min(cand), jnp.int32)

        if TPB == 1:
            return lwin_vec

        pub, tourn, shared = comm
        par = i & 1
        pub[pl.ds(0, L)] = jnp.full((L,), gmax, jnp.float32)
        pub[pl.ds(L, L)] = plsc.bitcast(lwin_vec, jnp.float32)
        pltpu.sync_copy(pub, shared.at[pl.ds((par * 16 + s) * SLOT, SLOT)])
        plsc.subcore_barrier()
        pltpu.sync_copy(
            shared.at[pl.ds((par * 16 + gg * TPB) * SLOT, TPB * SLOT)], tourn)

        bv = tourn[pl.ds(0, L)]
        bi = plsc.bitcast(tourn[pl.ds(L, L)], jnp.int32)
        for j in range(1, TPB):
            vj = tourn[pl.ds(j * SLOT, L)]
            ij = plsc.bitcast(tourn[pl.ds(j * SLOT + L, L)], jnp.int32)
            better = (vj > bv) | ((vj == bv) & (ij < bi))
            bv = jnp.where(better, vj, bv)
            bi = jnp.where(better, ij, bi)
        zeros = jnp.zeros((L,), jnp.int32)
        return jnp.minimum(jnp.maximum(bi, zeros),
                           jnp.full((L,), N - 1, jnp.int32))

    lax.fori_loop(0, K, outer, jnp.zeros((L,), jnp.int32))

    @pl.when(writer)
    def _():
        for j in range(KPT // L):
            iv = idxc[pl.ds(j * L, L)]
            for ci in range(C):
                outb[ci, pl.ds(j * L, L)] = plsc.load_gather(chans[ci], [iv])
        for ci in range(C):
            pltpu.sync_copy(outb.at[ci], out.at[b, ci, pl.ds(q * KPT, KPT)])


def _make_fps():
    mesh = plsc.VectorSubcoreMesh(core_axis_name="c", subcore_axis_name="s",
                                  num_cores=2, num_subcores=16)
    scratch = [
        pltpu.VMEM((N,), jnp.float32),
        pltpu.VMEM((N,), jnp.float32),
        pltpu.VMEM((N,), jnp.float32),
        pltpu.VMEM((N,), jnp.float32),
        pltpu.VMEM((N,), jnp.float32),
        pltpu.VMEM((N,), jnp.float32),
        pltpu.VMEM((NCHUNK,), jnp.float32),
        pltpu.VMEM((KPT,), jnp.int32),
        pltpu.VMEM((C, KPT), jnp.float32),
    ]
    if TPB > 1:
        scratch += [
            pltpu.VMEM((SLOT,), jnp.float32),
            pltpu.VMEM((TPB * SLOT,), jnp.float32),
            pltpu.VMEM_SHARED((2 * 16 * SLOT,), jnp.float32),
        ]
    return pl.kernel(
        _fps_body,
        out_type=jax.ShapeDtypeStruct((B, C, K), jnp.float32),
        mesh=mesh,
        compiler_params=pltpu.CompilerParams(needs_layout_passes=False),
        scratch_types=scratch,
    )


@jax.jit
def kernel(points):
    pts_t = jnp.transpose(points, (0, 2, 1))
    out_t = _make_fps()(pts_t)
    return jnp.transpose(out_t, (0, 2, 1))


# speedup vs baseline: 1.0773x; 1.0773x over previous
import jax
import jax.numpy as jnp
from jax import lax
from jax.experimental import pallas as pl
from jax.experimental.pallas import tpu as pltpu
from jax.experimental.pallas import tpu_sc as plsc

B, N, C = 8, 16384, 6
K = 1024
L = 16
TPB = 4
BPC = B // 2
NCHUNK = N // TPB
NV = NCHUNK // L
KPT = K // TPB
UNROLL = 8
SLOT = 2 * L
_PROBE_NO_EXCHANGE = False
INT_MAX = 2**31 - 1


def _fps_body(pts, out, ch0, ch1, ch2, ch3, ch4, ch5,
              dists, idxc, outb, *comm):
    c = lax.axis_index("c")
    s = lax.axis_index("s")
    gg = s // TPB
    q = s % TPB
    b = c * BPC + gg % BPC
    writer = gg < BPC
    doff = q * NCHUNK

    chans = (ch0, ch1, ch2, ch3, ch4, ch5)
    for ci in range(C):
        pltpu.sync_copy(pts.at[b, ci], chans[ci])

    inf_vec = jnp.full((L,), jnp.inf, jnp.float32)

    def init_body(v, carry):
        dists[pl.ds(v * L, L)] = inf_vec
        return carry

    lax.fori_loop(0, NV, init_body, 0)

    lane = lax.iota(jnp.int32, L)

    def outer(i, far_vec):
        cx = plsc.load_gather(ch0, [far_vec])
        cy = plsc.load_gather(ch1, [far_vec])
        cz = plsc.load_gather(ch2, [far_vec])

        @pl.when((i // KPT) == q)
        def _():
            plsc.store_scatter(idxc, [jnp.full((L,), i % KPT, jnp.int32)],
                               far_vec, mask=(lane == 0))

        best0 = jnp.full((L,), -1.0, jnp.float32)
        bidx0 = jnp.zeros((L,), jnp.int32)

        @plsc.parallel_loop(0, NCHUNK, step=L, unroll=UNROLL,
                            carry=(best0, bidx0))
        def inner(o, carry):
            best, bidx = carry
            xv = ch0[pl.ds(doff + o, L)]
            yv = ch1[pl.ds(doff + o, L)]
            zv = ch2[pl.ds(doff + o, L)]
            dx = xv - cx
            dy = yv - cy
            dz = zv - cz
            t = (dx * dx + dy * dy) + dz * dz
            dv = jnp.minimum(dists[pl.ds(o, L)], t)
            dists[pl.ds(o, L)] = dv
            m = dv > best
            best = jnp.maximum(best, dv)
            bidx = jnp.where(m, jnp.full((L,), doff + o, jnp.int32), bidx)
            return best, bidx

        best, bidx = inner
        bidx = bidx + lane

        gmax = jnp.max(best)
        cand = jnp.where(best == gmax, bidx, jnp.full((L,), INT_MAX, jnp.int32))
        lwin_vec = jnp.full((L,), jnp.min(cand), jnp.int32)

        if TPB == 1 or _PROBE_NO_EXCHANGE:
            return lwin_vec

        pub, tourn, shared = comm
        par = i & 1
        pub[pl.ds(0, L)] = jnp.full((L,), gmax, jnp.float32)
        pub[pl.ds(L, L)] = plsc.bitcast(lwin_vec, jnp.float32)
        pltpu.sync_copy(pub, shared.at[pl.ds((par * 16 + s) * SLOT, SLOT)])
        plsc.subcore_barrier()
        pltpu.sync_copy(
            shared.at[pl.ds((par * 16 + gg * TPB) * SLOT, TPB * SLOT)], tourn)

        bv = tourn[pl.ds(0, L)]
        bi = plsc.bitcast(tourn[pl.ds(L, L)], jnp.int32)
        for j in range(1, TPB):
            vj = tourn[pl.ds(j * SLOT, L)]
            ij = plsc.bitcast(tourn[pl.ds(j * SLOT + L, L)], jnp.int32)
            better = (vj > bv) | ((vj == bv) & (ij < bi))
            bv = jnp.where(better, vj, bv)
            bi = jnp.where(better, ij, bi)
        zeros = jnp.zeros((L,), jnp.int32)
        return jnp.minimum(jnp.maximum(bi, zeros),
                           jnp.full((L,), N - 1, jnp.int32))

    lax.fori_loop(0, K, outer, jnp.zeros((L,), jnp.int32))

    @pl.when(writer)
    def _():
        for j in range(KPT // L):
            iv = idxc[pl.ds(j * L, L)]
            for ci in range(C):
                outb[ci, pl.ds(j * L, L)] = plsc.load_gather(chans[ci], [iv])
        for ci in range(C):
            pltpu.sync_copy(outb.at[ci], out.at[b, ci, pl.ds(q * KPT, KPT)])


def _make_fps():
    mesh = plsc.VectorSubcoreMesh(core_axis_name="c", subcore_axis_name="s",
                                  num_cores=2, num_subcores=16)
    scratch = [
        pltpu.VMEM((N,), jnp.float32),
        pltpu.VMEM((N,), jnp.float32),
        pltpu.VMEM((N,), jnp.float32),
        pltpu.VMEM((N,), jnp.float32),
        pltpu.VMEM((N,), jnp.float32),
        pltpu.VMEM((N,), jnp.float32),
        pltpu.VMEM((NCHUNK,), jnp.float32),
        pltpu.VMEM((KPT,), jnp.int32),
        pltpu.VMEM((C, KPT), jnp.float32),
    ]
    if TPB > 1:
        scratch += [
            pltpu.VMEM((SLOT,), jnp.float32),
            pltpu.VMEM((TPB * SLOT,), jnp.float32),
            pltpu.VMEM_SHARED((2 * 16 * SLOT,), jnp.float32),
        ]
    return pl.kernel(
        _fps_body,
        out_type=jax.ShapeDtypeStruct((B, C, K), jnp.float32),
        mesh=mesh,
        compiler_params=pltpu.CompilerParams(needs_layout_passes=False),
        scratch_types=scratch,
    )


@jax.jit
def kernel(points):
    pts_t = jnp.transpose(points, (0, 2, 1))
    out_t = _make_fps()(pts_t)
    return jnp.transpose(out_t, (0, 2, 1))


# speedup vs baseline: 1.0785x; 1.0010x over previous
import jax
import jax.numpy as jnp
from jax import lax
from jax.experimental import pallas as pl
from jax.experimental.pallas import tpu as pltpu
from jax.experimental.pallas import tpu_sc as plsc

B, N, C = 8, 16384, 6
K = 1024
L = 16
TPB = 4
BPC = B // 2
NCHUNK = N // TPB
KPT = K // TPB
UNROLL = 8
SLOT = 2 * L
INT_MAX = 2**31 - 1


def _fps_body(pts, out, ch0, ch1, ch2, ch3, ch4, ch5,
              dists, idxc, outb, pub, tourn, shared):
    c = lax.axis_index("c")
    s = lax.axis_index("s")
    gg = s // TPB
    q = s % TPB
    b = c * BPC + gg
    doff = q * NCHUNK

    chans = (ch0, ch1, ch2, ch3, ch4, ch5)
    for ci in range(C):
        pltpu.sync_copy(pts.at[b, ci], chans[ci])

    inf_vec = jnp.full((L,), jnp.inf, jnp.float32)

    @plsc.parallel_loop(0, NCHUNK, step=L, unroll=UNROLL)
    def _init(v):
        dists[pl.ds(v, L)] = inf_vec

    lane = lax.iota(jnp.int32, L)

    def outer(i, far_vec):
        cx = plsc.load_gather(ch0, [far_vec])
        cy = plsc.load_gather(ch1, [far_vec])
        cz = plsc.load_gather(ch2, [far_vec])

        @pl.when((i // KPT) == q)
        def _():
            plsc.store_scatter(idxc, [jnp.full((L,), i % KPT, jnp.int32)],
                               far_vec, mask=(lane == 0))

        best0 = jnp.full((L,), -1.0, jnp.float32)
        bidx0 = jnp.zeros((L,), jnp.int32)

        @plsc.parallel_loop(0, NCHUNK, step=L, unroll=UNROLL,
                            carry=(best0, bidx0))
        def inner(o, carry):
            best, bidx = carry
            xv = ch0[pl.ds(doff + o, L)]
            yv = ch1[pl.ds(doff + o, L)]
            zv = ch2[pl.ds(doff + o, L)]
            dx = xv - cx
            dy = yv - cy
            dz = zv - cz
            t = (dx * dx + dy * dy) + dz * dz
            dv = jnp.minimum(dists[pl.ds(o, L)], t)
            dists[pl.ds(o, L)] = dv
            m = dv > best
            best = jnp.maximum(best, dv)
            bidx = jnp.where(m, jnp.full((L,), doff + o, jnp.int32), bidx)
            return best, bidx

        best, bidx = inner
        bidx = bidx + lane

        gmax = jnp.max(best)
        cand = jnp.where(best == gmax, bidx, jnp.full((L,), INT_MAX, jnp.int32))
        lwin_vec = jnp.full((L,), jnp.min(cand), jnp.int32)

        par = i & 1
        pub[pl.ds(0, L)] = jnp.full((L,), gmax, jnp.float32)
        pub[pl.ds(L, L)] = plsc.bitcast(lwin_vec, jnp.float32)
        pltpu.sync_copy(pub, shared.at[pl.ds((par * 16 + s) * SLOT, SLOT)])
        plsc.subcore_barrier()
        pltpu.sync_copy(
            shared.at[pl.ds((par * 16 + gg * TPB) * SLOT, TPB * SLOT)], tourn)

        bv = tourn[pl.ds(0, L)]
        bi = plsc.bitcast(tourn[pl.ds(L, L)], jnp.int32)
        for j in range(1, TPB):
            vj = tourn[pl.ds(j * SLOT, L)]
            ij = plsc.bitcast(tourn[pl.ds(j * SLOT + L, L)], jnp.int32)
            better = (vj > bv) | ((vj == bv) & (ij < bi))
            bv = jnp.where(better, vj, bv)
            bi = jnp.where(better, ij, bi)

        zeros = jnp.zeros((L,), jnp.int32)
        return jnp.minimum(jnp.maximum(bi, zeros),
                           jnp.full((L,), N - 1, jnp.int32))

    lax.fori_loop(0, K, outer, jnp.zeros((L,), jnp.int32))

    for j in range(KPT // L):
        iv = idxc[pl.ds(j * L, L)]
        for ci in range(C):
            outb[ci, pl.ds(j * L, L)] = plsc.load_gather(chans[ci], [iv])
    for ci in range(C):
        pltpu.sync_copy(outb.at[ci], out.at[b, ci, pl.ds(q * KPT, KPT)])


def _make_fps():
    mesh = plsc.VectorSubcoreMesh(core_axis_name="c", subcore_axis_name="s",
                                  num_cores=2, num_subcores=16)
    return pl.kernel(
        _fps_body,
        out_type=jax.ShapeDtypeStruct((B, C, K), jnp.float32),
        mesh=mesh,
        compiler_params=pltpu.CompilerParams(needs_layout_passes=False),
        scratch_types=[
            pltpu.VMEM((N,), jnp.float32),
            pltpu.VMEM((N,), jnp.float32),
            pltpu.VMEM((N,), jnp.float32),
            pltpu.VMEM((N,), jnp.float32),
            pltpu.VMEM((N,), jnp.float32),
            pltpu.VMEM((N,), jnp.float32),
            pltpu.VMEM((NCHUNK,), jnp.float32),
            pltpu.VMEM((KPT,), jnp.int32),
            pltpu.VMEM((C, KPT), jnp.float32),
            pltpu.VMEM((SLOT,), jnp.float32),
            pltpu.VMEM((TPB * SLOT,), jnp.float32),
            pltpu.VMEM_SHARED((2 * 16 * SLOT,), jnp.float32),
        ],
    )


@jax.jit
def kernel(points):
    pts_t = jnp.transpose(points, (0, 2, 1))
    out_t = _make_fps()(pts_t)
    return jnp.transpose(out_t, (0, 2, 1))


# speedup vs baseline: 1.0789x; 1.0004x over previous
import jax
import jax.numpy as jnp
from jax import lax
from jax.experimental import pallas as pl
from jax.experimental.pallas import tpu as pltpu
from jax.experimental.pallas import tpu_sc as plsc

B, N, C = 8, 16384, 6
K = 1024
L = 16
TPB = 4
BPC = B // 2
NCHUNK = N // TPB
KPT = K // TPB
UNROLL = 8
SLOT = 2 * L
INT_MAX = 2**31 - 1


def _fps_body(pts, out, ch0, ch1, ch2, ch3, ch4, ch5,
              dists, idxc, outb, pub, tourn, shared):
    c = lax.axis_index("c")
    s = lax.axis_index("s")
    gg = s // TPB
    q = s % TPB
    b = c * BPC + gg
    doff = q * NCHUNK

    chans = (ch0, ch1, ch2, ch3, ch4, ch5)
    for ci in range(C):
        pltpu.sync_copy(pts.at[b, ci], chans[ci])

    inf_vec = jnp.full((L,), jnp.inf, jnp.float32)

    @plsc.parallel_loop(0, NCHUNK, step=L, unroll=UNROLL)
    def _init(v):
        dists[pl.ds(v, L)] = inf_vec

    lane = lax.iota(jnp.int32, L)

    def outer(i, far_vec):
        cx = plsc.load_gather(ch0, [far_vec])
        cy = plsc.load_gather(ch1, [far_vec])
        cz = plsc.load_gather(ch2, [far_vec])

        @pl.when((i // KPT) == q)
        def _():
            plsc.store_scatter(idxc, [jnp.full((L,), i % KPT, jnp.int32)],
                               far_vec, mask=(lane == 0))

        best0 = jnp.full((L,), -1.0, jnp.float32)
        bidx0 = jnp.zeros((L,), jnp.int32)

        @plsc.parallel_loop(0, NCHUNK, step=L, unroll=UNROLL,
                            carry=(best0, bidx0))
        def inner(o, carry):
            best, bidx = carry
            xv = ch0[pl.ds(doff + o, L)]
            yv = ch1[pl.ds(doff + o, L)]
            zv = ch2[pl.ds(doff + o, L)]
            dx = xv - cx
            dy = yv - cy
            dz = zv - cz
            t = (dx * dx + dz * dz) + dy * dy
            dv = jnp.minimum(dists[pl.ds(o, L)], t)
            dists[pl.ds(o, L)] = dv
            m = dv > best
            best = jnp.maximum(best, dv)
            bidx = jnp.where(m, jnp.full((L,), doff + o, jnp.int32), bidx)
            return best, bidx

        best, bidx = inner
        bidx = bidx + lane

        gmax = jnp.max(best)
        cand = jnp.where(best == gmax, bidx, jnp.full((L,), INT_MAX, jnp.int32))
        lwin_vec = jnp.full((L,), jnp.min(cand), jnp.int32)

        par = i & 1
        pub[pl.ds(0, L)] = jnp.full((L,), gmax, jnp.float32)
        pub[pl.ds(L, L)] = plsc.bitcast(lwin_vec, jnp.float32)
        pltpu.sync_copy(pub, shared.at[pl.ds((par * 16 + s) * SLOT, SLOT)])
        plsc.subcore_barrier()
        pltpu.sync_copy(
            shared.at[pl.ds((par * 16 + gg * TPB) * SLOT, TPB * SLOT)], tourn)

        bv = tourn[pl.ds(0, L)]
        bi = plsc.bitcast(tourn[pl.ds(L, L)], jnp.int32)
        for j in range(1, TPB):
            vj = tourn[pl.ds(j * SLOT, L)]
            ij = plsc.bitcast(tourn[pl.ds(j * SLOT + L, L)], jnp.int32)
            better = (vj > bv) | ((vj == bv) & (ij < bi))
            bv = jnp.where(better, vj, bv)
            bi = jnp.where(better, ij, bi)

        zeros = jnp.zeros((L,), jnp.int32)
        return jnp.minimum(jnp.maximum(bi, zeros),
                           jnp.full((L,), N - 1, jnp.int32))

    lax.fori_loop(0, K, outer, jnp.zeros((L,), jnp.int32))

    for j in range(KPT // L):
        iv = idxc[pl.ds(j * L, L)]
        for ci in range(C):
            outb[ci, pl.ds(j * L, L)] = plsc.load_gather(chans[ci], [iv])
    for ci in range(C):
        pltpu.sync_copy(outb.at[ci], out.at[b, ci, pl.ds(q * KPT, KPT)])


def _make_fps():
    mesh = plsc.VectorSubcoreMesh(core_axis_name="c", subcore_axis_name="s",
                                  num_cores=2, num_subcores=16)
    return pl.kernel(
        _fps_body,
        out_type=jax.ShapeDtypeStruct((B, C, K), jnp.float32),
        mesh=mesh,
        compiler_params=pltpu.CompilerParams(needs_layout_passes=False),
        scratch_types=[
            pltpu.VMEM((N,), jnp.float32),
            pltpu.VMEM((N,), jnp.float32),
            pltpu.VMEM((N,), jnp.float32),
            pltpu.VMEM((N,), jnp.float32),
            pltpu.VMEM((N,), jnp.float32),
            pltpu.VMEM((N,), jnp.float32),
            pltpu.VMEM((NCHUNK,), jnp.float32),
            pltpu.VMEM((KPT,), jnp.int32),
            pltpu.VMEM((C, KPT), jnp.float32),
            pltpu.VMEM((SLOT,), jnp.float32),
            pltpu.VMEM((TPB * SLOT,), jnp.float32),
            pltpu.VMEM_SHARED((2 * 16 * SLOT,), jnp.float32),
        ],
    )


@jax.jit
def kernel(points):
    pts_t = jnp.transpose(points, (0, 2, 1))
    out_t = _make_fps()(pts_t)
    return jnp.transpose(out_t, (0, 2, 1))
